# baseline (speedup 1.0000x reference)
import functools

import jax
import jax.numpy as jnp
from jax import lax
from jax.experimental import pallas as pl
from jax.experimental.pallas import tpu as pltpu
from jax.experimental.pallas import tpu_sc as plsc

_B, _L, _D, _R, _P, _K, _N, _H = 32, 96, 256, 128, 24, 8, 4096, 4
_TEMP = 0.1
_LP = 104
_DH = _D // _H


def _matT(x, w):
    return lax.dot_general(x, w, (((1,), (1,)), ((), ())),
                           preferred_element_type=jnp.float32)


def _lnorm(x, g, b, eps=1e-5):
    m = jnp.mean(x, axis=-1, keepdims=True)
    v = jnp.mean((x - m) * (x - m), axis=-1, keepdims=True)
    return (x - m) / jnp.sqrt(v + eps) * g + b


def _gelu(x):
    return 0.5 * x * (1.0 + lax.erf(x * 0.7071067811865476))


def _r2(v):
    return v.reshape(1, -1)


_CH = 16


def _bdot(a, b, contract_a, contract_b):
    return lax.dot_general(a, b, (((contract_a,), (contract_b,)), ((0,), (0,))),
                           preferred_element_type=jnp.float32)


def _body_a(q_ref, cls_ref, mk_ref, sw_ref, *rest):
    ws = rest[:-2]
    wout_ref, iout_ref = rest[-2:]
    q = q_ref[...]
    cls = cls_ref[...]
    x = jnp.concatenate(
        [jnp.broadcast_to(cls[None], (_CH, 1, _D)), q,
         jnp.zeros((_CH, _LP - _L - 1, _D), jnp.float32)], axis=1)
    x = x.reshape(_CH * _LP, _D)
    kbias = jnp.where(
        lax.broadcasted_iota(jnp.int32, (1, 1, _LP), 2) < (_L + 1), 0.0, -1e9)
    (qkvW, qkvb, outW, outb, g1, b1, l1W, l1b, l2W, l2b, g2, b2) = (
        ws[j][...] for j in range(12))
    y = _matT(x, qkvW) + qkvb
    heads = []
    for h in range(_H):
        qh = y[:, h * _DH:(h + 1) * _DH].reshape(_CH, _LP, _DH)
        kh = y[:, _D + h * _DH:_D + (h + 1) * _DH].reshape(_CH, _LP, _DH)
        vh = y[:, 2 * _D + h * _DH:2 * _D + (h + 1) * _DH].reshape(
            _CH, _LP, _DH)
        s = _bdot(qh, kh, 2, 2) * 0.125
        e = jnp.exp(s + kbias)
        p = e * (1.0 / jnp.sum(e, axis=-1, keepdims=True))
        heads.append(_bdot(p, vh, 2, 1))
    o = jnp.concatenate(heads, axis=-1).reshape(_CH * _LP, _D)
    a = _matT(o, outW) + outb
    x = _lnorm(x + a, g1, b1)
    f = _matT(_gelu(_matT(x, l1W) + l1b), l2W) + l2b
    x = _lnorm(x + f, g2, b2)
    (qkvW, qkvb, outW, outb, g1, b1, l1W, l1b, l2W, l2b, g2, b2) = (
        ws[12 + j][...] for j in range(12))
    x0 = x.reshape(_CH, _LP, _D)[:, 0, :]
    ykv = _matT(x, qkvW[_D:, :]) + qkvb[:, _D:]
    yq0 = _matT(x0, qkvW[:_D, :]) + qkvb[:, :_D]
    kbias0 = jnp.where(
        lax.broadcasted_iota(jnp.int32, (1, _LP), 1) < (_L + 1), 0.0, -1e9)
    heads0 = []
    for h in range(_H):
        qh0 = yq0[:, h * _DH:(h + 1) * _DH]
        kh = ykv[:, h * _DH:(h + 1) * _DH].reshape(_CH, _LP, _DH)
        vh = ykv[:, _D + h * _DH:_D + (h + 1) * _DH].reshape(_CH, _LP, _DH)
        s = jnp.sum(qh0[:, None, :] * kh, axis=2) * 0.125
        e = jnp.exp(s + kbias0)
        p = e * (1.0 / jnp.sum(e, axis=-1, keepdims=True))
        heads0.append(jnp.sum(p[:, :, None] * vh, axis=1))
    o0 = jnp.concatenate(heads0, axis=-1)
    a0 = _matT(o0, outW) + outb
    x0 = _lnorm(x0 + a0, g1, b1)
    f0 = _matT(_gelu(_matT(x0, l1W) + l1b), l2W) + l2b
    x0 = _lnorm(x0 + f0, g2, b2)
    (pW, pb, pg, pbb, v1W, v1b, vg, vb, v2W, v2b) = (
        ws[24 + j][...] for j in range(10))
    rep = _lnorm(_matT(x0, pW) + pb, pg, pbb)
    vmean = jnp.mean(q, axis=1)
    val = _matT(_gelu(_lnorm(_matT(vmean, v1W) + v1b, vg, vb)), v2W) + v2b
    sw = sw_ref[0, 0]
    qk = sw * rep + (1.0 - sw) * val
    qn = qk / (jnp.sqrt(jnp.sum(qk * qk, axis=-1, keepdims=True)) + 1e-8)
    mk = mk_ref[...]
    kn = mk / (jnp.sqrt(jnp.sum(mk * mk, axis=-1, keepdims=True)) + 1e-8)
    s = _matT(qn, kn)
    iota = lax.broadcasted_iota(jnp.int32, (_CH, _N), 1)
    col = lax.broadcasted_iota(jnp.int32, (_CH, 128), 1)
    vals = jnp.zeros((_CH, 128), jnp.float32)
    idxs = jnp.zeros((_CH, 128), jnp.int32)
    for k in range(_K):
        m = jnp.max(s, axis=-1, keepdims=True)
        ind = jnp.min(jnp.where(s >= m, iota, _N), axis=-1, keepdims=True)
        vals = jnp.where(col == k, m, vals)
        idxs = jnp.where(col == k, ind, idxs)
        s = jnp.where(iota == ind, -1e30, s)
    logits = jnp.where(col < _K, vals * (1.0 / _TEMP), -1e30)
    e = jnp.exp(logits)
    wout_ref[...] = e * (1.0 / jnp.sum(e, axis=-1, keepdims=True))
    iout_ref[...] = idxs


def _run_a(query, cls2, mk, sw2, wlist):
    full = lambda a: pl.BlockSpec(a.shape, lambda i: (0,) * a.ndim)
    in_specs = ([pl.BlockSpec((_CH, _L, _D), lambda i: (i, 0, 0)),
                 full(cls2), full(mk), full(sw2)]
                + [full(w) for w in wlist])
    out_specs = [pl.BlockSpec((_CH, 128), lambda i: (i, 0))] * 2
    out_shape = [jax.ShapeDtypeStruct((_B, 128), jnp.float32),
                 jax.ShapeDtypeStruct((_B, 128), jnp.int32)]
    return pl.pallas_call(
        _body_a, grid=(_B // _CH,), in_specs=in_specs, out_specs=out_specs,
        out_shape=out_shape)(query, cls2, mk, sw2, *wlist)


def _sc_gather(table, idx):
    info = plsc.get_sparse_core_info()
    nw = info.num_cores * info.num_subcores
    bpw = (_B * _K * _P) // nw
    width = _D
    mesh = plsc.VectorSubcoreMesh(core_axis_name="c", subcore_axis_name="s")

    @functools.partial(
        pl.kernel, mesh=mesh,
        out_type=jax.ShapeDtypeStruct((_B * _K * _P, width), jnp.float32),
        scratch_types=[
            pltpu.VMEM((bpw,), jnp.int32),
            pltpu.VMEM((bpw, width), jnp.float32),
            pltpu.SemaphoreType.DMA,
        ],
    )
    def k(table_hbm, idx_hbm, out_hbm, idx_v, rows_v, sem):
        wid = lax.axis_index("s") * info.num_cores + lax.axis_index("c")
        base = wid * bpw
        pltpu.sync_copy(idx_hbm.at[pl.ds(base, bpw)], idx_v)
        pltpu.async_copy(table_hbm.at[idx_v], rows_v, sem).wait()
        pltpu.sync_copy(rows_v, out_hbm.at[pl.ds(base, bpw)])

    return k(table, idx)


def _body_c(q_ref, rv_ref, w_ref, *rest):
    ws = rest[:-3]
    out_ref, gate_ref, match_ref = rest[-3:]
    (cW, cb, cg, cbb, rW, rb, rg, rbb, m1W, m1b, m2W, m2b,
     g1W, g1b, g2W, g2b, caW, cab, coW, cob, oW, ob) = (r[...] for r in ws)
    q = q_ref[...]
    q2 = q.reshape(_B * _L, _D)
    rv = rv_ref[...]
    rp2 = jnp.mean(rv, axis=1)
    cur_pooled = jnp.mean(q, axis=1)
    cur_enc = _gelu(_lnorm(_matT(cur_pooled, cW) + cb, cg, cbb))
    ret_enc2 = _gelu(_lnorm(_matT(rp2, rW) + rb, rg, rbb))
    cur_rep = jnp.broadcast_to(
        cur_enc[:, None, :], (_B, _K, _D)).reshape(_B * _K, _D)
    mi = jnp.concatenate([cur_rep, ret_enc2], axis=1)
    mh = _gelu(_matT(mi, m1W) + m1b)
    mlogit = jnp.sum(mh * m2W, axis=-1, keepdims=True) + m2b
    match = 1.0 / (1.0 + jnp.exp(-mlogit))
    match_bk = match.reshape(_B, _K)
    w8 = w_ref[:, 0:_K]
    comb = w8 * match_bk
    comb = comb / (jnp.sum(comb, axis=-1, keepdims=True) + 1e-8)
    rp3 = rp2.reshape(_B, _K, _D)
    wref = jnp.sum(comb[:, :, None] * rp3, axis=1)
    wref_enc = _gelu(_lnorm(_matT(wref, rW) + rb, rg, rbb))
    gi = jnp.concatenate([cur_enc, wref_enc], axis=1)
    gh = _gelu(_matT(gi, g1W) + g1b)
    gate = 1.0 / (1.0 + jnp.exp(
        -(jnp.sum(gh * g2W, axis=-1, keepdims=True) + g2b)))
    qy = _matT(q2, caW[0:_D, :]) + cab[:, 0:_D]
    ky = _matT(ret_enc2, caW[_D:2 * _D, :]) + cab[:, _D:2 * _D]
    vy = _matT(ret_enc2, caW[2 * _D:3 * _D, :]) + cab[:, 2 * _D:3 * _D]
    outs = []
    for h in range(_H):
        qh = qy[:, h * _DH:(h + 1) * _DH].reshape(_B, _L, _DH)
        kh = ky[:, h * _DH:(h + 1) * _DH].reshape(_B, _K, _DH)
        vh = vy[:, h * _DH:(h + 1) * _DH].reshape(_B, _K, _DH)
        s = _bdot(qh, kh, 2, 2) * 0.125
        s = s - jnp.max(s, axis=-1, keepdims=True)
        e = jnp.exp(s)
        p = e / jnp.sum(e, axis=-1, keepdims=True)
        outs.append(_bdot(p, vh, 2, 1))
    o = jnp.concatenate(outs, axis=-1).reshape(_B * _L, _D)
    attn = (_matT(o, coW) + cob).reshape(_B, _L, _D)
    g3 = gate.reshape(_B, 1, 1)
    fused = g3 * q + (1.0 - g3) * attn
    cat = jnp.concatenate([q, fused], axis=-1).reshape(_B * _L, 2 * _D)
    out_ref[...] = (_matT(cat, oW) + ob).reshape(_B, _L, _D)
    col128 = lax.broadcasted_iota(jnp.int32, (_B, 128), 1)
    gate_ref[...] = jnp.where(col128 == 0, gate, 0.0)
    match_ref[...] = jnp.concatenate(
        [match_bk, jnp.zeros((_B, 128 - _K), jnp.float32)], axis=1)


def _run_c(query, rv, wpad, wlist):
    out_shape = [jax.ShapeDtypeStruct((_B, _L, _D), jnp.float32),
                 jax.ShapeDtypeStruct((_B, 128), jnp.float32),
                 jax.ShapeDtypeStruct((_B, 128), jnp.float32)]
    return pl.pallas_call(_body_c, out_shape=out_shape)(
        query, rv, wpad, *wlist)


def kernel(query, memory_keys, memory_values, params):
    p = params
    wlist_a = []
    for lyr in p["enc_layers"]:
        wlist_a += [lyr["qkv"]["W"], _r2(lyr["qkv"]["b"]),
                    lyr["out"]["W"], _r2(lyr["out"]["b"]),
                    _r2(lyr["ln1"]["g"]), _r2(lyr["ln1"]["b"]),
                    lyr["lin1"]["W"], _r2(lyr["lin1"]["b"]),
                    lyr["lin2"]["W"], _r2(lyr["lin2"]["b"]),
                    _r2(lyr["ln2"]["g"]), _r2(lyr["ln2"]["b"])]
    wlist_a += [p["proj_lin"]["W"], _r2(p["proj_lin"]["b"]),
                _r2(p["proj_ln"]["g"]), _r2(p["proj_ln"]["b"]),
                p["val1"]["W"], _r2(p["val1"]["b"]),
                _r2(p["val_ln"]["g"]), _r2(p["val_ln"]["b"]),
                p["val2"]["W"], _r2(p["val2"]["b"])]
    cls2 = p["cls"].reshape(1, _D)
    sw2 = p["shape_w"].reshape(1, 1)
    wpad, ipad = _run_a(query, cls2, memory_keys, sw2, wlist_a)
    idx = ipad[:, :_K].reshape(-1)
    idx24 = (idx[:, None] * _P
             + jnp.arange(_P, dtype=jnp.int32)[None, :]).reshape(-1)

    table = memory_values.reshape(_N * _P, _D)
    rv = _sc_gather(table, idx24).reshape(_B * _K, _P, _D)

    wlist_c = [p["cur_lin"]["W"], _r2(p["cur_lin"]["b"]),
               _r2(p["cur_ln"]["g"]), _r2(p["cur_ln"]["b"]),
               p["ret_lin"]["W"], _r2(p["ret_lin"]["b"]),
               _r2(p["ret_ln"]["g"]), _r2(p["ret_ln"]["b"]),
               p["match1"]["W"], _r2(p["match1"]["b"]),
               p["match2"]["W"], _r2(p["match2"]["b"]),
               p["gate1"]["W"], _r2(p["gate1"]["b"]),
               p["gate2"]["W"], _r2(p["gate2"]["b"]),
               p["ca_qkv"]["W"], _r2(p["ca_qkv"]["b"]),
               p["ca_out"]["W"], _r2(p["ca_out"]["b"]),
               p["out_proj"]["W"], _r2(p["out_proj"]["b"])]
    fused_out, gate_pad, match_pad = _run_c(query, rv, wpad, wlist_c)
    return fused_out, gate_pad[:, :1], match_pad[:, :_K]


# speedup vs baseline: 1.0642x; 1.0642x over previous
import functools

import jax
import jax.numpy as jnp
from jax import lax
from jax.experimental import pallas as pl
from jax.experimental.pallas import tpu as pltpu
from jax.experimental.pallas import tpu_sc as plsc

_B, _L, _D, _R, _P, _K, _N, _H = 32, 96, 256, 128, 24, 8, 4096, 4
_TEMP = 0.1
_LP = 104
_DH = _D // _H


def _matT(x, w):
    return lax.dot_general(x, w, (((1,), (1,)), ((), ())),
                           preferred_element_type=jnp.float32)


def _lnorm(x, g, b, eps=1e-5):
    m = jnp.mean(x, axis=-1, keepdims=True)
    v = jnp.mean((x - m) * (x - m), axis=-1, keepdims=True)
    return (x - m) / jnp.sqrt(v + eps) * g + b


def _gelu(x):
    return 0.5 * x * (1.0 + lax.erf(x * 0.7071067811865476))


def _r2(v):
    return v.reshape(1, -1)


_CH = 32


def _bdot(a, b, contract_a, contract_b):
    return lax.dot_general(a, b, (((contract_a,), (contract_b,)), ((0,), (0,))),
                           preferred_element_type=jnp.float32)


def _body_a(q_ref, cls_ref, mk_ref, sw_ref, *rest):
    ws = rest[:-2]
    wout_ref, iout_ref = rest[-2:]
    q = q_ref[...]
    cls = cls_ref[...]
    x = jnp.concatenate(
        [jnp.broadcast_to(cls[None], (_CH, 1, _D)), q,
         jnp.zeros((_CH, _LP - _L - 1, _D), jnp.float32)], axis=1)
    x = x.reshape(_CH * _LP, _D)
    kbias = jnp.where(
        lax.broadcasted_iota(jnp.int32, (1, 1, _LP), 2) < (_L + 1), 0.0, -1e9)
    (qkvW, qkvb, outW, outb, g1, b1, l1W, l1b, l2W, l2b, g2, b2) = (
        ws[j][...] for j in range(12))
    y = _matT(x, qkvW) + qkvb
    heads = []
    for h in range(_H):
        qh = y[:, h * _DH:(h + 1) * _DH].reshape(_CH, _LP, _DH)
        kh = y[:, _D + h * _DH:_D + (h + 1) * _DH].reshape(_CH, _LP, _DH)
        vh = y[:, 2 * _D + h * _DH:2 * _D + (h + 1) * _DH].reshape(
            _CH, _LP, _DH)
        s = _bdot(qh, kh, 2, 2) * 0.125
        e = jnp.exp(s + kbias)
        p = e * (1.0 / jnp.sum(e, axis=-1, keepdims=True))
        heads.append(_bdot(p, vh, 2, 1))
    o = jnp.concatenate(heads, axis=-1).reshape(_CH * _LP, _D)
    a = _matT(o, outW) + outb
    x = _lnorm(x + a, g1, b1)
    f = _matT(_gelu(_matT(x, l1W) + l1b), l2W) + l2b
    x = _lnorm(x + f, g2, b2)
    (qkvW, qkvb, outW, outb, g1, b1, l1W, l1b, l2W, l2b, g2, b2) = (
        ws[12 + j][...] for j in range(12))
    x0 = x.reshape(_CH, _LP, _D)[:, 0, :]
    ykv = _matT(x, qkvW[_D:, :]) + qkvb[:, _D:]
    yq0 = _matT(x0, qkvW[:_D, :]) + qkvb[:, :_D]
    kbias0 = jnp.where(
        lax.broadcasted_iota(jnp.int32, (1, _LP), 1) < (_L + 1), 0.0, -1e9)
    heads0 = []
    for h in range(_H):
        qh0 = yq0[:, h * _DH:(h + 1) * _DH]
        kh = ykv[:, h * _DH:(h + 1) * _DH].reshape(_CH, _LP, _DH)
        vh = ykv[:, _D + h * _DH:_D + (h + 1) * _DH].reshape(_CH, _LP, _DH)
        s = jnp.sum(qh0[:, None, :] * kh, axis=2) * 0.125
        e = jnp.exp(s + kbias0)
        p = e * (1.0 / jnp.sum(e, axis=-1, keepdims=True))
        heads0.append(jnp.sum(p[:, :, None] * vh, axis=1))
    o0 = jnp.concatenate(heads0, axis=-1)
    a0 = _matT(o0, outW) + outb
    x0 = _lnorm(x0 + a0, g1, b1)
    f0 = _matT(_gelu(_matT(x0, l1W) + l1b), l2W) + l2b
    x0 = _lnorm(x0 + f0, g2, b2)
    (pW, pb, pg, pbb, v1W, v1b, vg, vb, v2W, v2b) = (
        ws[24 + j][...] for j in range(10))
    rep = _lnorm(_matT(x0, pW) + pb, pg, pbb)
    vmean = jnp.mean(q, axis=1)
    val = _matT(_gelu(_lnorm(_matT(vmean, v1W) + v1b, vg, vb)), v2W) + v2b
    sw = sw_ref[0, 0]
    qk = sw * rep + (1.0 - sw) * val
    qn = qk / (jnp.sqrt(jnp.sum(qk * qk, axis=-1, keepdims=True)) + 1e-8)
    mk = mk_ref[...]
    kn = mk / (jnp.sqrt(jnp.sum(mk * mk, axis=-1, keepdims=True)) + 1e-8)
    s = _matT(qn, kn)
    iota = lax.broadcasted_iota(jnp.int32, (_CH, _N), 1)
    col = lax.broadcasted_iota(jnp.int32, (_CH, 128), 1)
    vals = jnp.zeros((_CH, 128), jnp.float32)
    idxs = jnp.zeros((_CH, 128), jnp.int32)
    for k in range(_K):
        m = jnp.max(s, axis=-1, keepdims=True)
        ind = jnp.min(jnp.where(s >= m, iota, _N), axis=-1, keepdims=True)
        vals = jnp.where(col == k, m, vals)
        idxs = jnp.where(col == k, ind, idxs)
        s = jnp.where(iota == ind, -1e30, s)
    logits = jnp.where(col < _K, vals * (1.0 / _TEMP), -1e30)
    e = jnp.exp(logits)
    wout_ref[...] = e * (1.0 / jnp.sum(e, axis=-1, keepdims=True))
    iout_ref[...] = idxs


def _run_a(query, cls2, mk, sw2, wlist):
    full = lambda a: pl.BlockSpec(a.shape, lambda i: (0,) * a.ndim)
    in_specs = ([pl.BlockSpec((_CH, _L, _D), lambda i: (i, 0, 0)),
                 full(cls2), full(mk), full(sw2)]
                + [full(w) for w in wlist])
    out_specs = [pl.BlockSpec((_CH, 128), lambda i: (i, 0))] * 2
    out_shape = [jax.ShapeDtypeStruct((_B, 128), jnp.float32),
                 jax.ShapeDtypeStruct((_B, 128), jnp.int32)]
    return pl.pallas_call(
        _body_a, grid=(_B // _CH,), in_specs=in_specs, out_specs=out_specs,
        out_shape=out_shape)(query, cls2, mk, sw2, *wlist)


def _sc_gather(table, idx):
    info = plsc.get_sparse_core_info()
    nw = info.num_cores * info.num_subcores
    bpw = (_B * _K * _P) // nw
    width = _D
    mesh = plsc.VectorSubcoreMesh(core_axis_name="c", subcore_axis_name="s")

    @functools.partial(
        pl.kernel, mesh=mesh,
        out_type=jax.ShapeDtypeStruct((_B * _K * _P, width), jnp.float32),
        scratch_types=[
            pltpu.VMEM((bpw,), jnp.int32),
            pltpu.VMEM((bpw, width), jnp.float32),
            pltpu.SemaphoreType.DMA,
        ],
    )
    def k(table_hbm, idx_hbm, out_hbm, idx_v, rows_v, sem):
        wid = lax.axis_index("s") * info.num_cores + lax.axis_index("c")
        base = wid * bpw
        pltpu.sync_copy(idx_hbm.at[pl.ds(base, bpw)], idx_v)
        pltpu.async_copy(table_hbm.at[idx_v], rows_v, sem).wait()
        pltpu.sync_copy(rows_v, out_hbm.at[pl.ds(base, bpw)])

    return k(table, idx)


def _body_c(q_ref, rv_ref, w_ref, *rest):
    ws = rest[:-3]
    out_ref, gate_ref, match_ref = rest[-3:]
    (cW, cb, cg, cbb, rW, rb, rg, rbb, m1W, m1b, m2W, m2b,
     g1W, g1b, g2W, g2b, caW, cab, coW, cob, oW, ob) = (r[...] for r in ws)
    q = q_ref[...]
    q2 = q.reshape(_B * _L, _D)
    rv = rv_ref[...]
    rp2 = jnp.mean(rv, axis=1)
    cur_pooled = jnp.mean(q, axis=1)
    cur_enc = _gelu(_lnorm(_matT(cur_pooled, cW) + cb, cg, cbb))
    ret_enc2 = _gelu(_lnorm(_matT(rp2, rW) + rb, rg, rbb))
    cur_rep = jnp.broadcast_to(
        cur_enc[:, None, :], (_B, _K, _D)).reshape(_B * _K, _D)
    mi = jnp.concatenate([cur_rep, ret_enc2], axis=1)
    mh = _gelu(_matT(mi, m1W) + m1b)
    mlogit = jnp.sum(mh * m2W, axis=-1, keepdims=True) + m2b
    match = 1.0 / (1.0 + jnp.exp(-mlogit))
    match_bk = match.reshape(_B, _K)
    w8 = w_ref[:, 0:_K]
    comb = w8 * match_bk
    comb = comb / (jnp.sum(comb, axis=-1, keepdims=True) + 1e-8)
    rp3 = rp2.reshape(_B, _K, _D)
    wref = jnp.sum(comb[:, :, None] * rp3, axis=1)
    wref_enc = _gelu(_lnorm(_matT(wref, rW) + rb, rg, rbb))
    gi = jnp.concatenate([cur_enc, wref_enc], axis=1)
    gh = _gelu(_matT(gi, g1W) + g1b)
    gate = 1.0 / (1.0 + jnp.exp(
        -(jnp.sum(gh * g2W, axis=-1, keepdims=True) + g2b)))
    qy = _matT(q2, caW[0:_D, :]) + cab[:, 0:_D]
    ky = _matT(ret_enc2, caW[_D:2 * _D, :]) + cab[:, _D:2 * _D]
    vy = _matT(ret_enc2, caW[2 * _D:3 * _D, :]) + cab[:, 2 * _D:3 * _D]
    outs = []
    for h in range(_H):
        qh = qy[:, h * _DH:(h + 1) * _DH].reshape(_B, _L, _DH)
        kh = ky[:, h * _DH:(h + 1) * _DH].reshape(_B, _K, _DH)
        vh = vy[:, h * _DH:(h + 1) * _DH].reshape(_B, _K, _DH)
        s = _bdot(qh, kh, 2, 2) * 0.125
        s = s - jnp.max(s, axis=-1, keepdims=True)
        e = jnp.exp(s)
        p = e / jnp.sum(e, axis=-1, keepdims=True)
        outs.append(_bdot(p, vh, 2, 1))
    o = jnp.concatenate(outs, axis=-1).reshape(_B * _L, _D)
    attn = (_matT(o, coW) + cob).reshape(_B, _L, _D)
    g3 = gate.reshape(_B, 1, 1)
    fused = g3 * q + (1.0 - g3) * attn
    cat = jnp.concatenate([q, fused], axis=-1).reshape(_B * _L, 2 * _D)
    out_ref[...] = (_matT(cat, oW) + ob).reshape(_B, _L, _D)
    col128 = lax.broadcasted_iota(jnp.int32, (_B, 128), 1)
    gate_ref[...] = jnp.where(col128 == 0, gate, 0.0)
    match_ref[...] = jnp.concatenate(
        [match_bk, jnp.zeros((_B, 128 - _K), jnp.float32)], axis=1)


def _run_c(query, rv, wpad, wlist):
    out_shape = [jax.ShapeDtypeStruct((_B, _L, _D), jnp.float32),
                 jax.ShapeDtypeStruct((_B, 128), jnp.float32),
                 jax.ShapeDtypeStruct((_B, 128), jnp.float32)]
    return pl.pallas_call(_body_c, out_shape=out_shape)(
        query, rv, wpad, *wlist)


def kernel(query, memory_keys, memory_values, params):
    p = params
    wlist_a = []
    for lyr in p["enc_layers"]:
        wlist_a += [lyr["qkv"]["W"], _r2(lyr["qkv"]["b"]),
                    lyr["out"]["W"], _r2(lyr["out"]["b"]),
                    _r2(lyr["ln1"]["g"]), _r2(lyr["ln1"]["b"]),
                    lyr["lin1"]["W"], _r2(lyr["lin1"]["b"]),
                    lyr["lin2"]["W"], _r2(lyr["lin2"]["b"]),
                    _r2(lyr["ln2"]["g"]), _r2(lyr["ln2"]["b"])]
    wlist_a += [p["proj_lin"]["W"], _r2(p["proj_lin"]["b"]),
                _r2(p["proj_ln"]["g"]), _r2(p["proj_ln"]["b"]),
                p["val1"]["W"], _r2(p["val1"]["b"]),
                _r2(p["val_ln"]["g"]), _r2(p["val_ln"]["b"]),
                p["val2"]["W"], _r2(p["val2"]["b"])]
    cls2 = p["cls"].reshape(1, _D)
    sw2 = p["shape_w"].reshape(1, 1)
    wpad, ipad = _run_a(query, cls2, memory_keys, sw2, wlist_a)
    idx = ipad[:, :_K].reshape(-1)
    idx24 = (idx[:, None] * _P
             + jnp.arange(_P, dtype=jnp.int32)[None, :]).reshape(-1)

    table = memory_values.reshape(_N * _P, _D)
    rv = _sc_gather(table, idx24).reshape(_B * _K, _P, _D)

    wlist_c = [p["cur_lin"]["W"], _r2(p["cur_lin"]["b"]),
               _r2(p["cur_ln"]["g"]), _r2(p["cur_ln"]["b"]),
               p["ret_lin"]["W"], _r2(p["ret_lin"]["b"]),
               _r2(p["ret_ln"]["g"]), _r2(p["ret_ln"]["b"]),
               p["match1"]["W"], _r2(p["match1"]["b"]),
               p["match2"]["W"], _r2(p["match2"]["b"]),
               p["gate1"]["W"], _r2(p["gate1"]["b"]),
               p["gate2"]["W"], _r2(p["gate2"]["b"]),
               p["ca_qkv"]["W"], _r2(p["ca_qkv"]["b"]),
               p["ca_out"]["W"], _r2(p["ca_out"]["b"]),
               p["out_proj"]["W"], _r2(p["out_proj"]["b"])]
    fused_out, gate_pad, match_pad = _run_c(query, rv, wpad, wlist_c)
    return fused_out, gate_pad[:, :1], match_pad[:, :_K]
